# baseline (speedup 1.0000x reference)
import functools

import jax
import jax.numpy as jnp
from jax import lax
from jax.experimental import pallas as pl
from jax.experimental.pallas import tpu as pltpu
from jax.experimental.pallas import tpu_sc as plsc

E = 8
TOPK = 2
BT = 256
BG = 512
NW = 32
GCH = 64


def _cast_body(w_ref, o_ref):
    o_ref[...] = w_ref[...].astype(jnp.bfloat16)


def _cast_bf16(w, BM):
    M, N = w.shape
    return pl.pallas_call(
        _cast_body,
        grid=(M // BM,),
        in_specs=[pl.BlockSpec((BM, N), lambda i: (i, 0))],
        out_specs=pl.BlockSpec((BM, N), lambda i: (i, 0)),
        out_shape=jax.ShapeDtypeStruct((M, N), jnp.bfloat16),
    )(w)


def _gate_body(x_ref, gw_ref, idx_ref, w_ref):
    logits = jnp.dot(x_ref[...], gw_ref[...],
                     preferred_element_type=jnp.float32)
    ecol = jax.lax.broadcasted_iota(jnp.int32, logits.shape, 1)
    m1 = jnp.max(logits, axis=1, keepdims=True)
    i1 = jnp.min(jnp.where(logits == m1, ecol, E), axis=1, keepdims=True)
    l2 = jnp.where(ecol == i1, -jnp.inf, logits)
    m2 = jnp.max(l2, axis=1, keepdims=True)
    i2 = jnp.min(jnp.where(l2 == m2, ecol, E), axis=1, keepdims=True)
    e2 = jnp.exp(m2 - m1)
    w0 = 1.0 / (1.0 + e2)
    w1 = e2 / (1.0 + e2)
    idx_ref[...] = jnp.concatenate([i1, i2], axis=1)
    w_ref[...] = jnp.concatenate([w0, w1], axis=1)


def _gating(x_flat, gate_w):
    T, D = x_flat.shape
    return pl.pallas_call(
        _gate_body,
        grid=(T // BG,),
        in_specs=[
            pl.BlockSpec((BG, D), lambda i: (i, 0)),
            pl.BlockSpec((D, E), lambda i: (0, 0)),
        ],
        out_specs=[
            pl.BlockSpec((BG, TOPK), lambda i: (i, 0)),
            pl.BlockSpec((BG, TOPK), lambda i: (i, 0)),
        ],
        out_shape=[
            jax.ShapeDtypeStruct((T, TOPK), jnp.int32),
            jax.ShapeDtypeStruct((T, TOPK), jnp.float32),
        ],
    )(x_flat, gate_w)


def _route_body(idx_ref, dest_ref, be_ref, exc_ref):
    T = idx_ref.shape[0]
    i1 = idx_ref[:, :1]
    i2 = idx_ref[:, 1:2]
    ecol = jax.lax.broadcasted_iota(jnp.int32, (T, E), 1)
    exc_ref[...] = (ecol == i1).astype(jnp.int32) + \
        (ecol == i2).astype(jnp.int32)
    s = 1
    while s < T:
        cur = exc_ref[...]
        exc_ref[...] = cur + jnp.concatenate(
            [jnp.zeros((s, E), jnp.int32), cur[:-s]], axis=0)
        s *= 2
    exc_ref[...] = exc_ref[...] - (
        (ecol == i1).astype(jnp.int32) + (ecol == i2).astype(jnp.int32))

    dest = jnp.zeros((T, TOPK), jnp.int32)
    bstart = jax.lax.broadcasted_iota(jnp.int32, be_ref.shape, 1) * BT
    be = jnp.zeros(be_ref.shape, jnp.int32)
    pe = jnp.int32(0)
    for e in range(E):
        if e > 0:
            be += (bstart >= pe).astype(jnp.int32)
        ma = (i1 == e).astype(jnp.int32)
        mb = (i2 == e).astype(jnp.int32)
        m = jnp.concatenate([ma, mb], axis=1)
        cs = jnp.concatenate([ma, ma + mb], axis=1)
        rank = exc_ref[:, e:e + 1] + cs - m
        dest = dest + m * (pe + rank)
        cnt = jnp.sum(m)
        pe = pe + ((cnt + BT - 1) // BT) * BT
    dest_ref[...] = dest
    be_ref[...] = be


def _route(idx, NB):
    T = idx.shape[0]
    return pl.pallas_call(
        _route_body,
        grid=(1,),
        in_specs=[pl.BlockSpec((T, TOPK), lambda i: (0, 0))],
        out_specs=[
            pl.BlockSpec((T, TOPK), lambda i: (0, 0)),
            pl.BlockSpec((1, NB), lambda i: (0, 0)),
        ],
        out_shape=[
            jax.ShapeDtypeStruct((T, TOPK), jnp.int32),
            jax.ShapeDtypeStruct((1, NB), jnp.int32),
        ],
        scratch_shapes=[pltpu.VMEM((T, E), jnp.int32)],
    )(idx)


def _ffn_body(be_ref, xs_ref, W1_ref, b1_ref, W2_ref, b2_ref, o_ref):
    h = jnp.dot(xs_ref[...], W1_ref[0], preferred_element_type=jnp.float32)
    h = h + b1_ref[0]
    h = 0.5 * h * (1.0 + jax.lax.erf(h * 0.7071067811865476))
    o = jnp.dot(h.astype(jnp.bfloat16), W2_ref[0],
                preferred_element_type=jnp.float32)
    o_ref[...] = (o + b2_ref[0]).astype(jnp.bfloat16)


def _grouped_ffn(xs, block_expert, W1, b1, W2, b2):
    P, D = xs.shape
    _, _, DFF = W1.shape
    NB = P // BT
    grid_spec = pltpu.PrefetchScalarGridSpec(
        num_scalar_prefetch=1,
        grid=(NB,),
        in_specs=[
            pl.BlockSpec((BT, D), lambda i, be: (i, 0)),
            pl.BlockSpec((1, D, DFF), lambda i, be: (be[i], 0, 0)),
            pl.BlockSpec((1, 1, DFF), lambda i, be: (be[i], 0, 0)),
            pl.BlockSpec((1, DFF, D), lambda i, be: (be[i], 0, 0)),
            pl.BlockSpec((1, 1, D), lambda i, be: (be[i], 0, 0)),
        ],
        out_specs=pl.BlockSpec((BT, D), lambda i, be: (i, 0)),
    )
    return pl.pallas_call(
        _ffn_body,
        grid_spec=grid_spec,
        out_shape=jax.ShapeDtypeStruct((P, D), jnp.bfloat16),
    )(block_expert, xs,
      _cast_bf16(W1.reshape(E * D, DFF), 512).reshape(E, D, DFF),
      b1.reshape(E, 1, DFF),
      _cast_bf16(W2.reshape(E * DFF, D), 2048).reshape(E, DFF, D),
      b2.reshape(E, 1, D))


def _sc_gather_rows(table, idx):
    V, D = table.shape
    P = idx.shape[0]
    rpw = P // NW
    nch = rpw // GCH
    mesh = plsc.VectorSubcoreMesh(core_axis_name="c", subcore_axis_name="s")

    @functools.partial(
        pl.kernel, mesh=mesh,
        out_type=jax.ShapeDtypeStruct((P, D), table.dtype),
        scratch_types=[
            pltpu.VMEM((GCH,), jnp.int32),
            pltpu.VMEM((GCH, D), table.dtype),
            pltpu.SemaphoreType.DMA,
        ],
    )
    def k(table_hbm, idx_hbm, out_hbm, idx_v, rows_v, sem):
        wid = lax.axis_index("s") * 2 + lax.axis_index("c")
        base = wid * rpw
        for c in range(nch):
            pltpu.sync_copy(idx_hbm.at[pl.ds(base + c * GCH, GCH)], idx_v)
            pltpu.async_copy(table_hbm.at[idx_v], rows_v, sem).wait()
            pltpu.sync_copy(rows_v, out_hbm.at[pl.ds(base + c * GCH, GCH)])

    return k(table, idx)


def _sc_scatter_rows(rows, jdst, G):
    P, D = rows.shape
    rpw = P // NW
    nch = rpw // GCH
    mesh = plsc.VectorSubcoreMesh(core_axis_name="c", subcore_axis_name="s")

    @functools.partial(
        pl.kernel, mesh=mesh,
        out_type=jax.ShapeDtypeStruct((G, D), rows.dtype),
        scratch_types=[
            pltpu.VMEM((GCH,), jnp.int32),
            pltpu.VMEM((GCH, D), rows.dtype),
            pltpu.SemaphoreType.DMA,
        ],
    )
    def k(rows_hbm, jdst_hbm, out_hbm, idx_v, buf_v, sem):
        wid = lax.axis_index("s") * 2 + lax.axis_index("c")
        base = wid * rpw
        for c in range(nch):
            pltpu.sync_copy(jdst_hbm.at[pl.ds(base + c * GCH, GCH)], idx_v)
            pltpu.sync_copy(rows_hbm.at[pl.ds(base + c * GCH, GCH)], buf_v)
            pltpu.async_copy(buf_v, out_hbm.at[idx_v], sem).wait()

    return k(rows, jdst)


def _combine_body(g0_ref, g1_ref, w_ref, out_ref):
    out_ref[...] = (g0_ref[...].astype(jnp.float32) * w_ref[:, :1] +
                    g1_ref[...].astype(jnp.float32) * w_ref[:, 1:2])


def _combine(g, w, T, D):
    BTC = 512
    return pl.pallas_call(
        _combine_body,
        grid=(T // BTC,),
        in_specs=[
            pl.BlockSpec((BTC, D), lambda i: (i, 0)),
            pl.BlockSpec((BTC, D), lambda i: (i + T // BTC, 0)),
            pl.BlockSpec((BTC, TOPK), lambda i: (i, 0)),
        ],
        out_specs=pl.BlockSpec((BTC, D), lambda i: (i, 0)),
        out_shape=jax.ShapeDtypeStruct((T, D), jnp.float32),
    )(g, g, w)


def kernel(x, gate_w, W1, b1, W2, b2):
    B, S, D = x.shape
    T = B * S
    x_flat = x.reshape(T, D)

    R = T * TOPK
    P = R + E * BT
    NB = P // BT
    idx, w = _gating(x_flat, gate_w)
    pos, be2 = _route(idx, NB)
    block_expert = be2.reshape(NB)
    dest = pos.reshape(R)
    ar = jnp.arange(R, dtype=jnp.int32)
    alt = (ar % TOPK) * T + ar // TOPK
    jdst = (R + (jnp.arange(P, dtype=jnp.int32) % 256)).at[dest].set(alt)
    row_token = jdst % T

    x16 = _cast_bf16(x_flat, 512)
    xi = jax.lax.bitcast_convert_type(
        x16.reshape(T, D // 2, 2), jnp.int32)
    xsi = _sc_gather_rows(xi, row_token)
    xs = jax.lax.bitcast_convert_type(xsi, jnp.bfloat16).reshape(P, D)
    o = _grouped_ffn(xs, block_expert, W1, b1, W2, b2)
    oi = jax.lax.bitcast_convert_type(
        o.reshape(P, D // 2, 2), jnp.int32)
    gi = _sc_scatter_rows(oi, jdst, R + 512)
    g = jax.lax.bitcast_convert_type(gi, jnp.bfloat16).reshape(R + 512, D)
    out = _combine(g, w, T, D)
    return out.reshape(B, S, D)


# speedup vs baseline: 2.5290x; 2.5290x over previous
import functools

import jax
import jax.numpy as jnp
from jax import lax
from jax.experimental import pallas as pl
from jax.experimental.pallas import tpu as pltpu
from jax.experimental.pallas import tpu_sc as plsc

E = 8
TOPK = 2
BT = 128
BG = 512
NW = 32
GCH = 48


def _cast_body(w_ref, o_ref):
    o_ref[...] = w_ref[...].astype(jnp.bfloat16)


def _cast_bf16(w, BM):
    M, N = w.shape
    return pl.pallas_call(
        _cast_body,
        grid=(M // BM,),
        in_specs=[pl.BlockSpec((BM, N), lambda i: (i, 0))],
        out_specs=pl.BlockSpec((BM, N), lambda i: (i, 0)),
        out_shape=jax.ShapeDtypeStruct((M, N), jnp.bfloat16),
    )(w)


def _gate_body(x_ref, gw_ref, idx_ref, w_ref):
    logits = jnp.dot(x_ref[...], gw_ref[...],
                     preferred_element_type=jnp.float32)
    ecol = jax.lax.broadcasted_iota(jnp.int32, logits.shape, 1)
    m1 = jnp.max(logits, axis=1, keepdims=True)
    i1 = jnp.min(jnp.where(logits == m1, ecol, E), axis=1, keepdims=True)
    l2 = jnp.where(ecol == i1, -jnp.inf, logits)
    m2 = jnp.max(l2, axis=1, keepdims=True)
    i2 = jnp.min(jnp.where(l2 == m2, ecol, E), axis=1, keepdims=True)
    e2 = jnp.exp(m2 - m1)
    w0 = 1.0 / (1.0 + e2)
    w1 = e2 / (1.0 + e2)
    idx_ref[...] = jnp.concatenate([i1, i2], axis=1)
    w_ref[...] = jnp.concatenate([w0, w1], axis=1)


def _gating(x_flat, gate_w):
    T, D = x_flat.shape
    return pl.pallas_call(
        _gate_body,
        grid=(T // BG,),
        in_specs=[
            pl.BlockSpec((BG, D), lambda i: (i, 0)),
            pl.BlockSpec((D, E), lambda i: (0, 0)),
        ],
        out_specs=[
            pl.BlockSpec((BG, TOPK), lambda i: (i, 0)),
            pl.BlockSpec((BG, TOPK), lambda i: (i, 0)),
        ],
        out_shape=[
            jax.ShapeDtypeStruct((T, TOPK), jnp.int32),
            jax.ShapeDtypeStruct((T, TOPK), jnp.float32),
        ],
    )(x_flat, gate_w)


def _route_body(idx_ref, dest_ref, be_ref, exc_ref):
    T = idx_ref.shape[0]
    i1 = idx_ref[:, :1]
    i2 = idx_ref[:, 1:2]
    ecol = jax.lax.broadcasted_iota(jnp.int32, (T, E), 1)
    exc_ref[...] = (ecol == i1).astype(jnp.int32) + \
        (ecol == i2).astype(jnp.int32)
    s = 1
    while s < T:
        cur = exc_ref[...]
        exc_ref[...] = cur + jnp.concatenate(
            [jnp.zeros((s, E), jnp.int32), cur[:-s]], axis=0)
        s *= 2
    exc_ref[...] = exc_ref[...] - (
        (ecol == i1).astype(jnp.int32) + (ecol == i2).astype(jnp.int32))

    dest = jnp.zeros((T, TOPK), jnp.int32)
    bstart = jax.lax.broadcasted_iota(jnp.int32, be_ref.shape, 1) * BT
    be = jnp.zeros(be_ref.shape, jnp.int32)
    pe = jnp.int32(0)
    for e in range(E):
        if e > 0:
            be += (bstart >= pe).astype(jnp.int32)
        ma = (i1 == e).astype(jnp.int32)
        mb = (i2 == e).astype(jnp.int32)
        m = jnp.concatenate([ma, mb], axis=1)
        cs = jnp.concatenate([ma, ma + mb], axis=1)
        rank = exc_ref[:, e:e + 1] + cs - m
        dest = dest + m * (pe + rank)
        cnt = jnp.sum(m)
        pe = pe + ((cnt + BT - 1) // BT) * BT
    dest_ref[...] = dest
    be_ref[...] = be


def _route(idx, NB):
    T = idx.shape[0]
    return pl.pallas_call(
        _route_body,
        grid=(1,),
        in_specs=[pl.BlockSpec((T, TOPK), lambda i: (0, 0))],
        out_specs=[
            pl.BlockSpec((T, TOPK), lambda i: (0, 0)),
            pl.BlockSpec((1, NB), lambda i: (0, 0)),
        ],
        out_shape=[
            jax.ShapeDtypeStruct((T, TOPK), jnp.int32),
            jax.ShapeDtypeStruct((1, NB), jnp.int32),
        ],
        scratch_shapes=[pltpu.VMEM((T, E), jnp.int32)],
    )(idx)


def _ffn_body(be_ref, xs_ref, W1_ref, b1_ref, W2_ref, b2_ref, o_ref):
    h = jnp.dot(xs_ref[...].astype(jnp.bfloat16),
                W1_ref[0].astype(jnp.bfloat16),
                preferred_element_type=jnp.float32)
    h = h + b1_ref[0]
    h = 0.5 * h * (1.0 + jax.lax.erf(h * 0.7071067811865476))
    o = jnp.dot(h.astype(jnp.bfloat16), W2_ref[0],
                preferred_element_type=jnp.float32)
    o_ref[...] = o + b2_ref[0]


def _grouped_ffn(xs, block_expert, W1, b1, W2, b2):
    P, D = xs.shape
    _, _, DFF = W1.shape
    NB = P // BT
    grid_spec = pltpu.PrefetchScalarGridSpec(
        num_scalar_prefetch=1,
        grid=(NB,),
        in_specs=[
            pl.BlockSpec((BT, D), lambda i, be: (i, 0)),
            pl.BlockSpec((1, D, DFF), lambda i, be: (be[i], 0, 0)),
            pl.BlockSpec((1, 1, DFF), lambda i, be: (be[i], 0, 0)),
            pl.BlockSpec((1, DFF, D), lambda i, be: (be[i], 0, 0)),
            pl.BlockSpec((1, 1, D), lambda i, be: (be[i], 0, 0)),
        ],
        out_specs=pl.BlockSpec((BT, D), lambda i, be: (i, 0)),
    )
    return pl.pallas_call(
        _ffn_body,
        grid_spec=grid_spec,
        out_shape=jax.ShapeDtypeStruct((P, D), jnp.float32),
    )(block_expert, xs, W1, b1.reshape(E, 1, DFF),
      _cast_bf16(W2.reshape(E * DFF, D), 2048).reshape(E, DFF, D),
      b2.reshape(E, 1, D))


def _sc_gather_rows(table, idx):
    V, D = table.shape
    P = idx.shape[0]
    rpw = P // NW
    nch = rpw // GCH
    mesh = plsc.VectorSubcoreMesh(core_axis_name="c", subcore_axis_name="s")

    @functools.partial(
        pl.kernel, mesh=mesh,
        out_type=jax.ShapeDtypeStruct((P, D), jnp.float32),
        scratch_types=[
            pltpu.VMEM((GCH,), jnp.int32),
            pltpu.VMEM((GCH, D), jnp.float32),
            pltpu.SemaphoreType.DMA,
        ],
    )
    def k(table_hbm, idx_hbm, out_hbm, idx_v, rows_v, sem):
        wid = lax.axis_index("s") * 2 + lax.axis_index("c")
        base = wid * rpw
        for c in range(nch):
            pltpu.sync_copy(idx_hbm.at[pl.ds(base + c * GCH, GCH)], idx_v)
            pltpu.async_copy(table_hbm.at[idx_v], rows_v, sem).wait()
            pltpu.sync_copy(rows_v, out_hbm.at[pl.ds(base + c * GCH, GCH)])

    return k(table, idx)


def _sc_scatter_rows(rows, jdst, G):
    P, D = rows.shape
    rpw = P // NW
    nch = rpw // GCH
    mesh = plsc.VectorSubcoreMesh(core_axis_name="c", subcore_axis_name="s")

    @functools.partial(
        pl.kernel, mesh=mesh,
        out_type=jax.ShapeDtypeStruct((G, D), jnp.float32),
        scratch_types=[
            pltpu.VMEM((GCH,), jnp.int32),
            pltpu.VMEM((GCH, D), jnp.float32),
            pltpu.SemaphoreType.DMA,
        ],
    )
    def k(rows_hbm, jdst_hbm, out_hbm, idx_v, buf_v, sem):
        wid = lax.axis_index("s") * 2 + lax.axis_index("c")
        base = wid * rpw
        for c in range(nch):
            pltpu.sync_copy(jdst_hbm.at[pl.ds(base + c * GCH, GCH)], idx_v)
            pltpu.sync_copy(rows_hbm.at[pl.ds(base + c * GCH, GCH)], buf_v)
            pltpu.async_copy(buf_v, out_hbm.at[idx_v], sem).wait()

    return k(rows, jdst)


def _combine_body(g0_ref, g1_ref, w_ref, out_ref):
    out_ref[...] = g0_ref[...] * w_ref[:, :1] + g1_ref[...] * w_ref[:, 1:2]


def _combine(g, w, T, D):
    BTC = 512
    return pl.pallas_call(
        _combine_body,
        grid=(T // BTC,),
        in_specs=[
            pl.BlockSpec((BTC, D), lambda i: (i, 0)),
            pl.BlockSpec((BTC, D), lambda i: (i + T // BTC, 0)),
            pl.BlockSpec((BTC, TOPK), lambda i: (i, 0)),
        ],
        out_specs=pl.BlockSpec((BTC, D), lambda i: (i, 0)),
        out_shape=jax.ShapeDtypeStruct((T, D), jnp.float32),
    )(g, g, w)


def kernel(x, gate_w, W1, b1, W2, b2):
    B, S, D = x.shape
    T = B * S
    x_flat = x.reshape(T, D)

    R = T * TOPK
    P = R + E * BT
    NB = P // BT
    idx, w = _gating(x_flat, gate_w)
    pos, be2 = _route(idx, NB)
    block_expert = be2.reshape(NB)
    dest = pos.reshape(R)
    ar = jnp.arange(R, dtype=jnp.int32)
    alt = (ar % TOPK) * T + ar // TOPK
    jdst = (R + (jnp.arange(P, dtype=jnp.int32) % 256)).at[dest].set(alt)
    row_token = jdst % T

    xs = _sc_gather_rows(x_flat, row_token)
    o = _grouped_ffn(xs, block_expert, W1, b1, W2, b2)
    g = _sc_scatter_rows(o, jdst, R + 512)
    out = _combine(g, w, T, D)
    return out.reshape(B, S, D)


# speedup vs baseline: 2.6095x; 1.0318x over previous
import functools

import jax
import jax.numpy as jnp
from jax import lax
from jax.experimental import pallas as pl
from jax.experimental.pallas import tpu as pltpu
from jax.experimental.pallas import tpu_sc as plsc

E = 8
TOPK = 2
BT = 256
BG = 512
NW = 32
GCH = 64


def _cast_body(w_ref, o_ref):
    o_ref[...] = w_ref[...].astype(jnp.bfloat16)


def _cast_bf16(w, BM):
    M, N = w.shape
    return pl.pallas_call(
        _cast_body,
        grid=(M // BM,),
        in_specs=[pl.BlockSpec((BM, N), lambda i: (i, 0))],
        out_specs=pl.BlockSpec((BM, N), lambda i: (i, 0)),
        out_shape=jax.ShapeDtypeStruct((M, N), jnp.bfloat16),
    )(w)


def _gate_body(x_ref, gw_ref, idx_ref, w_ref):
    logits = jnp.dot(x_ref[...], gw_ref[...],
                     preferred_element_type=jnp.float32)
    ecol = jax.lax.broadcasted_iota(jnp.int32, logits.shape, 1)
    m1 = jnp.max(logits, axis=1, keepdims=True)
    i1 = jnp.min(jnp.where(logits == m1, ecol, E), axis=1, keepdims=True)
    l2 = jnp.where(ecol == i1, -jnp.inf, logits)
    m2 = jnp.max(l2, axis=1, keepdims=True)
    i2 = jnp.min(jnp.where(l2 == m2, ecol, E), axis=1, keepdims=True)
    e2 = jnp.exp(m2 - m1)
    w0 = 1.0 / (1.0 + e2)
    w1 = e2 / (1.0 + e2)
    idx_ref[...] = jnp.concatenate([i1, i2], axis=1)
    w_ref[...] = jnp.concatenate([w0, w1], axis=1)


def _gating(x_flat, gate_w):
    T, D = x_flat.shape
    return pl.pallas_call(
        _gate_body,
        grid=(T // BG,),
        in_specs=[
            pl.BlockSpec((BG, D), lambda i: (i, 0)),
            pl.BlockSpec((D, E), lambda i: (0, 0)),
        ],
        out_specs=[
            pl.BlockSpec((BG, TOPK), lambda i: (i, 0)),
            pl.BlockSpec((BG, TOPK), lambda i: (i, 0)),
        ],
        out_shape=[
            jax.ShapeDtypeStruct((T, TOPK), jnp.int32),
            jax.ShapeDtypeStruct((T, TOPK), jnp.float32),
        ],
    )(x_flat, gate_w)


def _route_body(idx_ref, dest_ref, be_ref, exc_ref):
    T = idx_ref.shape[0]
    i1 = idx_ref[:, :1]
    i2 = idx_ref[:, 1:2]
    ecol = jax.lax.broadcasted_iota(jnp.int32, (T, E), 1)
    exc_ref[...] = (ecol == i1).astype(jnp.int32) + \
        (ecol == i2).astype(jnp.int32)
    s = 1
    while s < T:
        cur = exc_ref[...]
        exc_ref[...] = cur + jnp.concatenate(
            [jnp.zeros((s, E), jnp.int32), cur[:-s]], axis=0)
        s *= 2
    exc_ref[...] = exc_ref[...] - (
        (ecol == i1).astype(jnp.int32) + (ecol == i2).astype(jnp.int32))

    dest = jnp.zeros((T, TOPK), jnp.int32)
    bstart = jax.lax.broadcasted_iota(jnp.int32, be_ref.shape, 1) * BT
    be = jnp.zeros(be_ref.shape, jnp.int32)
    pe = jnp.int32(0)
    for e in range(E):
        if e > 0:
            be += (bstart >= pe).astype(jnp.int32)
        ma = (i1 == e).astype(jnp.int32)
        mb = (i2 == e).astype(jnp.int32)
        m = jnp.concatenate([ma, mb], axis=1)
        cs = jnp.concatenate([ma, ma + mb], axis=1)
        rank = exc_ref[:, e:e + 1] + cs - m
        dest = dest + m * (pe + rank)
        cnt = jnp.sum(m)
        pe = pe + ((cnt + BT - 1) // BT) * BT
    dest_ref[...] = dest
    be_ref[...] = be


def _route(idx, NB):
    T = idx.shape[0]
    return pl.pallas_call(
        _route_body,
        grid=(1,),
        in_specs=[pl.BlockSpec((T, TOPK), lambda i: (0, 0))],
        out_specs=[
            pl.BlockSpec((T, TOPK), lambda i: (0, 0)),
            pl.BlockSpec((1, NB), lambda i: (0, 0)),
        ],
        out_shape=[
            jax.ShapeDtypeStruct((T, TOPK), jnp.int32),
            jax.ShapeDtypeStruct((1, NB), jnp.int32),
        ],
        scratch_shapes=[pltpu.VMEM((T, E), jnp.int32)],
    )(idx)


def _ffn_body(be_ref, xs_ref, W1_ref, b1_ref, W2_ref, b2_ref, o_ref):
    h = jnp.dot(xs_ref[...].astype(jnp.bfloat16),
                W1_ref[0].astype(jnp.bfloat16),
                preferred_element_type=jnp.float32)
    h = h + b1_ref[0]
    h = 0.5 * h * (1.0 + jax.lax.erf(h * 0.7071067811865476))
    o = jnp.dot(h.astype(jnp.bfloat16), W2_ref[0],
                preferred_element_type=jnp.float32)
    o_ref[...] = o + b2_ref[0]


def _grouped_ffn(xs, block_expert, W1, b1, W2, b2):
    P, D = xs.shape
    _, _, DFF = W1.shape
    NB = P // BT
    grid_spec = pltpu.PrefetchScalarGridSpec(
        num_scalar_prefetch=1,
        grid=(NB,),
        in_specs=[
            pl.BlockSpec((BT, D), lambda i, be: (i, 0)),
            pl.BlockSpec((1, D, DFF), lambda i, be: (be[i], 0, 0)),
            pl.BlockSpec((1, 1, DFF), lambda i, be: (be[i], 0, 0)),
            pl.BlockSpec((1, DFF, D), lambda i, be: (be[i], 0, 0)),
            pl.BlockSpec((1, 1, D), lambda i, be: (be[i], 0, 0)),
        ],
        out_specs=pl.BlockSpec((BT, D), lambda i, be: (i, 0)),
    )
    return pl.pallas_call(
        _ffn_body,
        grid_spec=grid_spec,
        out_shape=jax.ShapeDtypeStruct((P, D), jnp.float32),
    )(block_expert, xs, W1, b1.reshape(E, 1, DFF),
      _cast_bf16(W2.reshape(E * DFF, D), 2048).reshape(E, DFF, D),
      b2.reshape(E, 1, D))


def _sc_gather_rows(table, idx):
    V, D = table.shape
    P = idx.shape[0]
    rpw = P // NW
    nch = rpw // GCH
    mesh = plsc.VectorSubcoreMesh(core_axis_name="c", subcore_axis_name="s")

    @functools.partial(
        pl.kernel, mesh=mesh,
        out_type=jax.ShapeDtypeStruct((P, D), jnp.float32),
        scratch_types=[
            pltpu.VMEM((GCH,), jnp.int32),
            pltpu.VMEM((GCH, D), jnp.float32),
            pltpu.SemaphoreType.DMA,
        ],
    )
    def k(table_hbm, idx_hbm, out_hbm, idx_v, rows_v, sem):
        wid = lax.axis_index("s") * 2 + lax.axis_index("c")
        base = wid * rpw
        for c in range(nch):
            pltpu.sync_copy(idx_hbm.at[pl.ds(base + c * GCH, GCH)], idx_v)
            pltpu.async_copy(table_hbm.at[idx_v], rows_v, sem).wait()
            pltpu.sync_copy(rows_v, out_hbm.at[pl.ds(base + c * GCH, GCH)])

    return k(table, idx)


def _sc_scatter_rows(rows, jdst, G):
    P, D = rows.shape
    rpw = P // NW
    nch = rpw // GCH
    mesh = plsc.VectorSubcoreMesh(core_axis_name="c", subcore_axis_name="s")

    @functools.partial(
        pl.kernel, mesh=mesh,
        out_type=jax.ShapeDtypeStruct((G, D), jnp.float32),
        scratch_types=[
            pltpu.VMEM((GCH,), jnp.int32),
            pltpu.VMEM((GCH, D), jnp.float32),
            pltpu.SemaphoreType.DMA,
        ],
    )
    def k(rows_hbm, jdst_hbm, out_hbm, idx_v, buf_v, sem):
        wid = lax.axis_index("s") * 2 + lax.axis_index("c")
        base = wid * rpw
        for c in range(nch):
            pltpu.sync_copy(jdst_hbm.at[pl.ds(base + c * GCH, GCH)], idx_v)
            pltpu.sync_copy(rows_hbm.at[pl.ds(base + c * GCH, GCH)], buf_v)
            pltpu.async_copy(buf_v, out_hbm.at[idx_v], sem).wait()

    return k(rows, jdst)


def _combine_body(g0_ref, g1_ref, w_ref, out_ref):
    out_ref[...] = g0_ref[...] * w_ref[:, :1] + g1_ref[...] * w_ref[:, 1:2]


def _combine(g, w, T, D):
    BTC = 512
    return pl.pallas_call(
        _combine_body,
        grid=(T // BTC,),
        in_specs=[
            pl.BlockSpec((BTC, D), lambda i: (i, 0)),
            pl.BlockSpec((BTC, D), lambda i: (i + T // BTC, 0)),
            pl.BlockSpec((BTC, TOPK), lambda i: (i, 0)),
        ],
        out_specs=pl.BlockSpec((BTC, D), lambda i: (i, 0)),
        out_shape=jax.ShapeDtypeStruct((T, D), jnp.float32),
    )(g, g, w)


def kernel(x, gate_w, W1, b1, W2, b2):
    B, S, D = x.shape
    T = B * S
    x_flat = x.reshape(T, D)

    R = T * TOPK
    P = R + E * BT
    NB = P // BT
    idx, w = _gating(x_flat, gate_w)
    pos, be2 = _route(idx, NB)
    block_expert = be2.reshape(NB)
    dest = pos.reshape(R)
    ar = jnp.arange(R, dtype=jnp.int32)
    alt = (ar % TOPK) * T + ar // TOPK
    jdst = (R + (jnp.arange(P, dtype=jnp.int32) % 256)).at[dest].set(alt)
    row_token = jdst % T

    xs = _sc_gather_rows(x_flat, row_token)
    o = _grouped_ffn(xs, block_expert, W1, b1, W2, b2)
    g = _sc_scatter_rows(o, jdst, R + 512)
    out = _combine(g, w, T, D)
    return out.reshape(B, S, D)
